# baseline (speedup 1.0000x reference)
import jax
import jax.numpy as jnp
from jax import lax
from jax.experimental import pallas as pl
from jax.experimental.pallas import tpu as pltpu
from jax.experimental.pallas import tpu_sc as plsc

N_NODES = 10000
N_EDGES = 160000
D = 256
NS = 16
NC = 2
NW = NC * NS
EPT = N_EDGES // NW
CPP = 4
SLAB = N_NODES * CPP
NQ = D // CPP
POS = N_EDGES * CPP
GP = 16000
NEG = POS // GP

_SC_PARAMS = pltpu.CompilerParams(needs_layout_passes=False)


def _deg_body(e2_hbm, zeros_hbm, deg_hbm, idx_v, hist):
    c = lax.axis_index("c")
    s = lax.axis_index("s")
    w = c * NS + s
    pltpu.sync_copy(zeros_hbm, hist)
    ones = jnp.ones((16,), jnp.float32)

    pltpu.sync_copy(e2_hbm.at[pl.ds(w * (2 * EPT), 2 * EPT)], idx_v)

    @pl.loop(0, 2 * EPT // 16)
    def _(j):
        iv = idx_v[pl.ds(j * 16, 16)]
        plsc.addupdate_scatter(hist, [iv], ones)

    pltpu.sync_copy(hist, deg_hbm.at[pl.ds(w * 2 * N_NODES, 2 * N_NODES)])


def _sc_degrees(e2, zeros2n):
    mesh = plsc.VectorSubcoreMesh(core_axis_name="c", subcore_axis_name="s")
    f = pl.kernel(
        _deg_body,
        out_type=jax.ShapeDtypeStruct((NW * 2 * N_NODES,), jnp.float32),
        mesh=mesh,
        compiler_params=_SC_PARAMS,
        scratch_types=[
            pltpu.VMEM((2 * EPT,), jnp.int32),
            pltpu.VMEM((2 * N_NODES,), jnp.float32),
        ],
    )
    return f(e2, zeros2n)


def _mm_body(x_ref, hist_ref, w_ref, out_ref):
    deg = jnp.sum(hist_ref[...], axis=1, keepdims=True)
    norm = lax.rsqrt(jnp.maximum(deg, 1.0))
    xb = x_ref[...] * norm
    out_ref[...] = jnp.dot(xb, w_ref[...], preferred_element_type=jnp.float32)


def _tc_matmul(x, hist_src, W):
    R = 400
    grid = (N_NODES // R,)
    return pl.pallas_call(
        _mm_body,
        grid=grid,
        in_specs=[
            pl.BlockSpec((R, D), lambda i: (i, 0)),
            pl.BlockSpec((R, NW), lambda i: (i, 0)),
            pl.BlockSpec((D, D), lambda i: (0, 0)),
        ],
        out_specs=pl.BlockSpec((R, D), lambda i: (i, 0)),
        out_shape=jax.ShapeDtypeStruct((N_NODES, D), jnp.float32),
    )(x, hist_src, W)


def _agg_body(featq_hbm, src4_hbm, dst4_hbm, zeros_hbm, out_hbm,
              slab, agg, s4, d4):
    c = lax.axis_index("c")
    s = lax.axis_index("s")
    w = c * NS + s

    for p in range(2):
        q = w * 2 + p
        pltpu.sync_copy(featq_hbm.at[pl.ds(q * SLAB, SLAB)], slab)
        pltpu.sync_copy(zeros_hbm, agg)

        @pl.loop(0, NEG)
        def _(g):
            pltpu.sync_copy(src4_hbm.at[pl.ds(g * GP, GP)], s4)
            pltpu.sync_copy(dst4_hbm.at[pl.ds(g * GP, GP)], d4)

            @pl.loop(0, GP // 16)
            def _(t):
                pv = s4[pl.ds(t * 16, 16)]
                mv = plsc.load_gather(slab, [pv])
                dv = d4[pl.ds(t * 16, 16)]
                plsc.addupdate_scatter(agg, [dv], mv)

        pltpu.sync_copy(agg, out_hbm.at[pl.ds(q * SLAB, SLAB)])


def _sc_aggregate(featq, src4, dst4, zeros4):
    mesh = plsc.VectorSubcoreMesh(core_axis_name="c", subcore_axis_name="s")
    f = pl.kernel(
        _agg_body,
        out_type=jax.ShapeDtypeStruct((NQ * SLAB,), jnp.float32),
        mesh=mesh,
        compiler_params=_SC_PARAMS,
        scratch_types=[
            pltpu.VMEM((SLAB,), jnp.float32),
            pltpu.VMEM((SLAB,), jnp.float32),
            pltpu.VMEM((GP,), jnp.int32),
            pltpu.VMEM((GP,), jnp.int32),
        ],
    )
    return f(featq, src4, dst4, zeros4)


def _epi_body(agg_ref, hist_ref, b_ref, out_ref):
    deg = jnp.sum(hist_ref[...], axis=1, keepdims=True)
    norm = lax.rsqrt(jnp.maximum(deg, 1.0))
    y = agg_ref[...] * norm + b_ref[...]
    out_ref[...] = jnp.maximum(y, 0.0)


def _tc_epilogue(agg, hist_dst, b2):
    R = 400
    grid = (N_NODES // R,)
    return pl.pallas_call(
        _epi_body,
        grid=grid,
        in_specs=[
            pl.BlockSpec((R, D), lambda i: (i, 0)),
            pl.BlockSpec((R, NW), lambda i: (i, 0)),
            pl.BlockSpec((1, D), lambda i: (0, 0)),
        ],
        out_specs=pl.BlockSpec((R, D), lambda i: (i, 0)),
        out_shape=jax.ShapeDtypeStruct((N_NODES, D), jnp.float32),
    )(agg, hist_dst, b2)


def kernel(x, edge_index, W, b):
    src = edge_index[0]
    dst = edge_index[1]

    e2 = jnp.concatenate(
        [src.reshape(NW, EPT), dst.reshape(NW, EPT) + N_NODES],
        axis=1).reshape(-1)
    zeros2n = jnp.zeros((2 * N_NODES,), jnp.float32)
    part = _sc_degrees(e2, zeros2n).reshape(NW, 2, N_NODES)
    hist_src = part[:, 0, :].T
    hist_dst = part[:, 1, :].T

    feat = _tc_matmul(x, hist_src, W)
    featq = feat.reshape(N_NODES, NQ, CPP).transpose(1, 0, 2).reshape(-1)

    lane4 = jnp.arange(CPP, dtype=jnp.int32)
    src4 = (src[:, None] * CPP + lane4[None, :]).reshape(-1)
    dst4 = (dst[:, None] * CPP + lane4[None, :]).reshape(-1)
    zeros4 = jnp.zeros((SLAB,), jnp.float32)

    aggq = _sc_aggregate(featq, src4, dst4, zeros4)
    agg = aggq.reshape(NQ, N_NODES, CPP).transpose(1, 0, 2).reshape(
        N_NODES, D)
    return _tc_epilogue(agg, hist_dst, b.reshape(1, D))


# speedup vs baseline: 1.0425x; 1.0425x over previous
import jax
import jax.numpy as jnp
from jax import lax
from jax.experimental import pallas as pl
from jax.experimental.pallas import tpu as pltpu
from jax.experimental.pallas import tpu_sc as plsc

N_NODES = 10000
N_EDGES = 160000
D = 256
NS = 16
NC = 2
NW = NC * NS
EPT = N_EDGES // NW
CPP = 4
SLAB = N_NODES * CPP
NQ = D // CPP
POS = N_EDGES * CPP
GP = 16000
NEG = POS // GP

_SC_PARAMS = pltpu.CompilerParams(needs_layout_passes=False)


def _deg_body(e2_hbm, zeros_hbm, deg_hbm, idx_v, hist):
    c = lax.axis_index("c")
    s = lax.axis_index("s")
    w = c * NS + s
    pltpu.sync_copy(zeros_hbm, hist)
    ones = jnp.ones((16,), jnp.float32)

    pltpu.sync_copy(e2_hbm.at[pl.ds(w * (2 * EPT), 2 * EPT)], idx_v)

    @pl.loop(0, 2 * EPT // 16)
    def _(j):
        iv = idx_v[pl.ds(j * 16, 16)]
        plsc.addupdate_scatter(hist, [iv], ones)

    pltpu.sync_copy(hist, deg_hbm.at[pl.ds(w * 2 * N_NODES, 2 * N_NODES)])


def _sc_degrees(e2, zeros2n):
    mesh = plsc.VectorSubcoreMesh(core_axis_name="c", subcore_axis_name="s")
    f = pl.kernel(
        _deg_body,
        out_type=jax.ShapeDtypeStruct((NW * 2 * N_NODES,), jnp.float32),
        mesh=mesh,
        compiler_params=_SC_PARAMS,
        scratch_types=[
            pltpu.VMEM((2 * EPT,), jnp.int32),
            pltpu.VMEM((2 * N_NODES,), jnp.float32),
        ],
    )
    return f(e2, zeros2n)


def _mm_body(x_ref, hist_ref, w_ref, out_ref):
    deg = jnp.sum(hist_ref[...], axis=1, keepdims=True)
    norm = lax.rsqrt(jnp.maximum(deg, 1.0))
    xb = x_ref[...] * norm
    out_ref[...] = jnp.dot(xb, w_ref[...], preferred_element_type=jnp.float32)


def _tc_matmul(x, hist_src, W):
    R = 400
    grid = (N_NODES // R,)
    return pl.pallas_call(
        _mm_body,
        grid=grid,
        in_specs=[
            pl.BlockSpec((R, D), lambda i: (i, 0)),
            pl.BlockSpec((R, NW), lambda i: (i, 0)),
            pl.BlockSpec((D, D), lambda i: (0, 0)),
        ],
        out_specs=pl.BlockSpec((R, D), lambda i: (i, 0)),
        out_shape=jax.ShapeDtypeStruct((N_NODES, D), jnp.float32),
    )(x, hist_src, W)


def _agg_body(featq_hbm, pk4_hbm, zeros_hbm, out_hbm, slab, agg, p4):
    c = lax.axis_index("c")
    s = lax.axis_index("s")
    w = c * NS + s

    for p in range(2):
        q = w * 2 + p
        pltpu.sync_copy(featq_hbm.at[pl.ds(q * SLAB, SLAB)], slab)
        pltpu.sync_copy(zeros_hbm, agg)

        @pl.loop(0, NEG)
        def _(g):
            pltpu.sync_copy(pk4_hbm.at[pl.ds(g * GP, GP)], p4)

            @pl.loop(0, GP // 16, unroll=8)
            def _(t):
                pk = p4[pl.ds(t * 16, 16)]
                pv = lax.bitwise_and(pk, jnp.int32(0xFFFF))
                dv = lax.shift_right_logical(pk, jnp.int32(16))
                mv = plsc.load_gather(slab, [pv])
                plsc.addupdate_scatter(agg, [dv], mv)

        pltpu.sync_copy(agg, out_hbm.at[pl.ds(q * SLAB, SLAB)])


def _sc_aggregate(featq, pk4, zeros4):
    mesh = plsc.VectorSubcoreMesh(core_axis_name="c", subcore_axis_name="s")
    f = pl.kernel(
        _agg_body,
        out_type=jax.ShapeDtypeStruct((NQ * SLAB,), jnp.float32),
        mesh=mesh,
        compiler_params=_SC_PARAMS,
        scratch_types=[
            pltpu.VMEM((SLAB,), jnp.float32),
            pltpu.VMEM((SLAB,), jnp.float32),
            pltpu.VMEM((GP,), jnp.int32),
        ],
    )
    return f(featq, pk4, zeros4)


def _epi_body(agg_ref, hist_ref, b_ref, out_ref):
    deg = jnp.sum(hist_ref[...], axis=1, keepdims=True)
    norm = lax.rsqrt(jnp.maximum(deg, 1.0))
    y = agg_ref[...] * norm + b_ref[...]
    out_ref[...] = jnp.maximum(y, 0.0)


def _tc_epilogue(agg, hist_dst, b2):
    R = 400
    grid = (N_NODES // R,)
    return pl.pallas_call(
        _epi_body,
        grid=grid,
        in_specs=[
            pl.BlockSpec((R, D), lambda i: (i, 0)),
            pl.BlockSpec((R, NW), lambda i: (i, 0)),
            pl.BlockSpec((1, D), lambda i: (0, 0)),
        ],
        out_specs=pl.BlockSpec((R, D), lambda i: (i, 0)),
        out_shape=jax.ShapeDtypeStruct((N_NODES, D), jnp.float32),
    )(agg, hist_dst, b2)


def kernel(x, edge_index, W, b):
    src = edge_index[0]
    dst = edge_index[1]

    e2 = jnp.concatenate(
        [src.reshape(NW, EPT), dst.reshape(NW, EPT) + N_NODES],
        axis=1).reshape(-1)
    zeros2n = jnp.zeros((2 * N_NODES,), jnp.float32)
    part = _sc_degrees(e2, zeros2n).reshape(NW, 2, N_NODES)
    hist_src = part[:, 0, :].T
    hist_dst = part[:, 1, :].T

    feat = _tc_matmul(x, hist_src, W)
    featq = feat.reshape(N_NODES, NQ, CPP).transpose(1, 0, 2).reshape(-1)

    lane4 = jnp.arange(CPP, dtype=jnp.int32)
    src4 = (src[:, None] * CPP + lane4[None, :]).reshape(-1)
    dst4 = (dst[:, None] * CPP + lane4[None, :]).reshape(-1)
    pk4 = src4 | (dst4 << 16)
    zeros4 = jnp.zeros((SLAB,), jnp.float32)

    aggq = _sc_aggregate(featq, pk4, zeros4)
    agg = aggq.reshape(NQ, N_NODES, CPP).transpose(1, 0, 2).reshape(
        N_NODES, D)
    return _tc_epilogue(agg, hist_dst, b.reshape(1, D))


# speedup vs baseline: 1.2494x; 1.1985x over previous
import jax
import jax.numpy as jnp
from jax import lax
from jax.experimental import pallas as pl
from jax.experimental.pallas import tpu as pltpu
from jax.experimental.pallas import tpu_sc as plsc

N_NODES = 10000
N_EDGES = 160000
D = 256
NS = 16
NC = 2
NW = NC * NS
EPT = N_EDGES // NW
CPP = 4
SLAB = N_NODES * CPP
NQ = D // CPP
POS = N_EDGES * CPP
GP = 16000
NEG = POS // GP

_SC_PARAMS = pltpu.CompilerParams(needs_layout_passes=False)


def _deg_body(e2_hbm, zeros_hbm, deg_hbm, idx_v, hist):
    c = lax.axis_index("c")
    s = lax.axis_index("s")
    w = c * NS + s
    pltpu.sync_copy(zeros_hbm, hist)
    ones = jnp.ones((16,), jnp.float32)

    pltpu.sync_copy(e2_hbm.at[pl.ds(w * (2 * EPT), 2 * EPT)], idx_v)

    @pl.loop(0, 2 * EPT // 16)
    def _(j):
        iv = idx_v[pl.ds(j * 16, 16)]
        plsc.addupdate_scatter(hist, [iv], ones)

    pltpu.sync_copy(hist, deg_hbm.at[pl.ds(w * 2 * N_NODES, 2 * N_NODES)])


def _sc_degrees(e2, zeros2n):
    mesh = plsc.VectorSubcoreMesh(core_axis_name="c", subcore_axis_name="s")
    f = pl.kernel(
        _deg_body,
        out_type=jax.ShapeDtypeStruct((NW * 2 * N_NODES,), jnp.float32),
        mesh=mesh,
        compiler_params=_SC_PARAMS,
        scratch_types=[
            pltpu.VMEM((2 * EPT,), jnp.int32),
            pltpu.VMEM((2 * N_NODES,), jnp.float32),
        ],
    )
    return f(e2, zeros2n)


def _mm_body(x_ref, hist_ref, w_ref, out_ref):
    deg = jnp.sum(hist_ref[...], axis=1, keepdims=True)
    norm = lax.rsqrt(jnp.maximum(deg, 1.0))
    xb = x_ref[...] * norm
    y = jnp.dot(xb, w_ref[...], preferred_element_type=jnp.float32)
    for q in range(NQ):
        out_ref[q] = y[:, q * CPP:(q + 1) * CPP]


def _tc_matmul(x, hist_src, W):
    R = 400
    grid = (N_NODES // R,)
    return pl.pallas_call(
        _mm_body,
        grid=grid,
        in_specs=[
            pl.BlockSpec((R, D), lambda i: (i, 0)),
            pl.BlockSpec((R, NW), lambda i: (i, 0)),
            pl.BlockSpec((D, D), lambda i: (0, 0)),
        ],
        out_specs=pl.BlockSpec((NQ, R, CPP), lambda i: (0, i, 0)),
        out_shape=jax.ShapeDtypeStruct((NQ, N_NODES, CPP), jnp.float32),
    )(x, hist_src, W)


def _agg_body(featq_hbm, pk4_hbm, zeros_hbm, out_hbm, slab, agg, p4):
    c = lax.axis_index("c")
    s = lax.axis_index("s")
    w = c * NS + s

    for p in range(2):
        q = w * 2 + p
        pltpu.sync_copy(featq_hbm.at[pl.ds(q * SLAB, SLAB)], slab)
        pltpu.sync_copy(zeros_hbm, agg)

        @pl.loop(0, NEG)
        def _(g):
            pltpu.sync_copy(pk4_hbm.at[pl.ds(g * GP, GP)], p4)

            @pl.loop(0, GP // 16, unroll=8)
            def _(t):
                pk = p4[pl.ds(t * 16, 16)]
                pv = lax.bitwise_and(pk, jnp.int32(0xFFFF))
                dv = lax.shift_right_logical(pk, jnp.int32(16))
                mv = plsc.load_gather(slab, [pv])
                plsc.addupdate_scatter(agg, [dv], mv)

        pltpu.sync_copy(agg, out_hbm.at[pl.ds(q * SLAB, SLAB)])


def _sc_aggregate(featq, pk4, zeros4):
    mesh = plsc.VectorSubcoreMesh(core_axis_name="c", subcore_axis_name="s")
    f = pl.kernel(
        _agg_body,
        out_type=jax.ShapeDtypeStruct((NQ * SLAB,), jnp.float32),
        mesh=mesh,
        compiler_params=_SC_PARAMS,
        scratch_types=[
            pltpu.VMEM((SLAB,), jnp.float32),
            pltpu.VMEM((SLAB,), jnp.float32),
            pltpu.VMEM((GP,), jnp.int32),
        ],
    )
    return f(featq, pk4, zeros4)


def _epi_body(agg_ref, hist_ref, b_ref, out_ref):
    deg = jnp.sum(hist_ref[...], axis=1, keepdims=True)
    norm = lax.rsqrt(jnp.maximum(deg, 1.0))
    y = jnp.concatenate([agg_ref[q] for q in range(NQ)], axis=1)
    y = y * norm + b_ref[...]
    out_ref[...] = jnp.maximum(y, 0.0)


def _tc_epilogue(agg, hist_dst, b2):
    R = 400
    grid = (N_NODES // R,)
    return pl.pallas_call(
        _epi_body,
        grid=grid,
        in_specs=[
            pl.BlockSpec((NQ, R, CPP), lambda i: (0, i, 0)),
            pl.BlockSpec((R, NW), lambda i: (i, 0)),
            pl.BlockSpec((1, D), lambda i: (0, 0)),
        ],
        out_specs=pl.BlockSpec((R, D), lambda i: (i, 0)),
        out_shape=jax.ShapeDtypeStruct((N_NODES, D), jnp.float32),
    )(agg, hist_dst, b2)


def _e2_body(s_ref, d_ref, o_ref):
    o_ref[:, 0, :] = s_ref[...]
    o_ref[:, 1, :] = d_ref[...] + N_NODES


def _tc_prep_e2(src_r, dst_r):
    grid = (NW // 8,)
    return pl.pallas_call(
        _e2_body,
        grid=grid,
        in_specs=[
            pl.BlockSpec((8, EPT), lambda i: (i, 0)),
            pl.BlockSpec((8, EPT), lambda i: (i, 0)),
        ],
        out_specs=pl.BlockSpec((8, 2, EPT), lambda i: (i, 0, 0)),
        out_shape=jax.ShapeDtypeStruct((NW, 2, EPT), jnp.int32),
    )(src_r, dst_r)


def _pk_body(s_ref, d_ref, pk_ref):
    EB = s_ref.shape[0]
    l = lax.broadcasted_iota(jnp.int32, (EB, CPP), 1)
    sv = s_ref[...] * CPP + l
    dv = d_ref[...] * CPP + l
    pk_ref[...] = jnp.bitwise_or(sv, dv << 16)


def _tc_prep_pk(src_c, dst_c):
    EB = 8000
    grid = (N_EDGES // EB,)
    return pl.pallas_call(
        _pk_body,
        grid=grid,
        in_specs=[
            pl.BlockSpec((EB, 1), lambda i: (i, 0)),
            pl.BlockSpec((EB, 1), lambda i: (i, 0)),
        ],
        out_specs=pl.BlockSpec((EB, CPP), lambda i: (i, 0)),
        out_shape=jax.ShapeDtypeStruct((N_EDGES, CPP), jnp.int32),
    )(src_c, dst_c)


def _tp_body(part_ref, hs_ref, hd_ref):
    y = part_ref[...]
    hs_ref[...] = y[:, 0, :].T
    hd_ref[...] = y[:, 1, :].T


def _tc_hist_transpose(part):
    return pl.pallas_call(
        _tp_body,
        grid=(1,),
        in_specs=[pl.BlockSpec((NW, 2, N_NODES), lambda i: (0, 0, 0))],
        out_specs=[
            pl.BlockSpec((N_NODES, NW), lambda i: (0, 0)),
            pl.BlockSpec((N_NODES, NW), lambda i: (0, 0)),
        ],
        out_shape=[
            jax.ShapeDtypeStruct((N_NODES, NW), jnp.float32),
            jax.ShapeDtypeStruct((N_NODES, NW), jnp.float32),
        ],
    )(part)


def kernel(x, edge_index, W, b):
    src = edge_index[0]
    dst = edge_index[1]

    e2 = _tc_prep_e2(src.reshape(NW, EPT), dst.reshape(NW, EPT)).reshape(-1)
    zeros2n = jnp.zeros((2 * N_NODES,), jnp.float32)
    part = _sc_degrees(e2, zeros2n).reshape(NW, 2, N_NODES)
    hist_src, hist_dst = _tc_hist_transpose(part)

    featq = _tc_matmul(x, hist_src, W).reshape(-1)
    pk4 = _tc_prep_pk(src.reshape(N_EDGES, 1),
                      dst.reshape(N_EDGES, 1)).reshape(-1)
    zeros4 = jnp.zeros((SLAB,), jnp.float32)

    aggq = _sc_aggregate(featq, pk4, zeros4)
    agg = aggq.reshape(NQ, N_NODES, CPP)
    return _tc_epilogue(agg, hist_dst, b.reshape(1, D))


# speedup vs baseline: 2.3079x; 1.8472x over previous
import jax
import jax.numpy as jnp
from jax import lax
from jax.experimental import pallas as pl
from jax.experimental.pallas import tpu as pltpu
from jax.experimental.pallas import tpu_sc as plsc

N_NODES = 10000
N_EDGES = 160000
D = 256
NS = 16
NC = 2
NW = NC * NS
EPT = N_EDGES // NW
CPP = 4
SLAB = N_NODES * CPP
NQ = D // CPP
POS = N_EDGES * CPP
GP = 16000
NEG = POS // GP
N_PAD = 10240
SLAB_P = CPP * N_PAD
R2 = 1280

_SC_PARAMS = pltpu.CompilerParams(needs_layout_passes=False)


def _deg_body(e2_hbm, zeros_hbm, deg_hbm, idx_v, hist):
    c = lax.axis_index("c")
    s = lax.axis_index("s")
    w = c * NS + s
    pltpu.sync_copy(zeros_hbm, hist)
    ones = jnp.ones((16,), jnp.float32)

    pltpu.sync_copy(e2_hbm.at[pl.ds(w * (2 * EPT), 2 * EPT)], idx_v)

    @pl.loop(0, 2 * EPT // 16)
    def _(j):
        iv = idx_v[pl.ds(j * 16, 16)]
        plsc.addupdate_scatter(hist, [iv], ones)

    pltpu.sync_copy(hist, deg_hbm.at[pl.ds(w * 2 * N_NODES, 2 * N_NODES)])


def _sc_degrees(e2, zeros2n):
    mesh = plsc.VectorSubcoreMesh(core_axis_name="c", subcore_axis_name="s")
    f = pl.kernel(
        _deg_body,
        out_type=jax.ShapeDtypeStruct((NW * 2 * N_NODES,), jnp.float32),
        mesh=mesh,
        compiler_params=_SC_PARAMS,
        scratch_types=[
            pltpu.VMEM((2 * EPT,), jnp.int32),
            pltpu.VMEM((2 * N_NODES,), jnp.float32),
        ],
    )
    return f(e2, zeros2n)


def _mm_body(x_ref, hist_ref, w_ref, out_ref):
    deg = jnp.sum(hist_ref[...], axis=1, keepdims=True)
    norm = lax.rsqrt(jnp.maximum(deg, 1.0))
    xb = x_ref[...] * norm
    y = jnp.dot(xb, w_ref[...], preferred_element_type=jnp.float32)
    out_ref[...] = y.T.reshape(NQ, CPP, y.shape[0])


def _unused_mm():
    pass


def _tc_matmul(x, hist_src, W):
    grid = (N_PAD // R2,)
    return pl.pallas_call(
        _mm_body,
        grid=grid,
        in_specs=[
            pl.BlockSpec((R2, D), lambda i: (i, 0)),
            pl.BlockSpec((R2, NW), lambda i: (i, 0)),
            pl.BlockSpec((D, D), lambda i: (0, 0)),
        ],
        out_specs=pl.BlockSpec((NQ, CPP, R2), lambda i: (0, 0, i)),
        out_shape=jax.ShapeDtypeStruct((NQ, CPP, N_PAD), jnp.float32),
    )(x, hist_src, W)


def _agg_body(featq_hbm, pk4_hbm, zeros_hbm, out_hbm, slab, agg, p4):
    c = lax.axis_index("c")
    s = lax.axis_index("s")
    w = c * NS + s

    for p in range(2):
        q = w * 2 + p
        pltpu.sync_copy(featq_hbm.at[pl.ds(q * SLAB_P, SLAB_P)], slab)
        pltpu.sync_copy(zeros_hbm, agg)

        @pl.loop(0, NEG)
        def _(g):
            pltpu.sync_copy(pk4_hbm.at[pl.ds(g * GP, GP)], p4)

            @pl.loop(0, GP // 16, unroll=8)
            def _(t):
                pk = p4[pl.ds(t * 16, 16)]
                pv = lax.bitwise_and(pk, jnp.int32(0xFFFF))
                dv = lax.shift_right_logical(pk, jnp.int32(16))
                mv = plsc.load_gather(slab, [pv])
                plsc.addupdate_scatter(agg, [dv], mv)

        pltpu.sync_copy(agg, out_hbm.at[pl.ds(q * SLAB_P, SLAB_P)])


def _sc_aggregate(featq, pk4, zeros4):
    mesh = plsc.VectorSubcoreMesh(core_axis_name="c", subcore_axis_name="s")
    f = pl.kernel(
        _agg_body,
        out_type=jax.ShapeDtypeStruct((NQ * SLAB_P,), jnp.float32),
        mesh=mesh,
        compiler_params=_SC_PARAMS,
        scratch_types=[
            pltpu.VMEM((SLAB_P,), jnp.float32),
            pltpu.VMEM((SLAB_P,), jnp.float32),
            pltpu.VMEM((GP,), jnp.int32),
        ],
    )
    return f(featq, pk4, zeros4)


def _epi_body(agg_ref, hist_ref, b_ref, out_ref):
    deg = jnp.sum(hist_ref[...], axis=1, keepdims=True)
    norm = lax.rsqrt(jnp.maximum(deg, 1.0))
    a = agg_ref[...]
    y = a.reshape(D, a.shape[2]).T
    y = y * norm + b_ref[...]
    out_ref[...] = jnp.maximum(y, 0.0)


def _tc_epilogue(agg, hist_dst, b2):
    grid = (N_PAD // R2,)
    return pl.pallas_call(
        _epi_body,
        grid=grid,
        in_specs=[
            pl.BlockSpec((NQ, CPP, R2), lambda i: (0, 0, i)),
            pl.BlockSpec((R2, NW), lambda i: (i, 0)),
            pl.BlockSpec((1, D), lambda i: (0, 0)),
        ],
        out_specs=pl.BlockSpec((R2, D), lambda i: (i, 0)),
        out_shape=jax.ShapeDtypeStruct((N_NODES, D), jnp.float32),
    )(agg, hist_dst, b2)


def _e2_body(s_ref, d_ref, o_ref):
    o_ref[:, 0, :] = s_ref[...]
    o_ref[:, 1, :] = d_ref[...] + N_NODES


def _tc_prep_e2(src_r, dst_r):
    grid = (NW // 8,)
    return pl.pallas_call(
        _e2_body,
        grid=grid,
        in_specs=[
            pl.BlockSpec((8, EPT), lambda i: (i, 0)),
            pl.BlockSpec((8, EPT), lambda i: (i, 0)),
        ],
        out_specs=pl.BlockSpec((8, 2, EPT), lambda i: (i, 0, 0)),
        out_shape=jax.ShapeDtypeStruct((NW, 2, EPT), jnp.int32),
    )(src_r, dst_r)


E8 = N_EDGES // 8


def _pk_body(s_ref, d_ref, pk_ref):
    lN = lax.broadcasted_iota(jnp.int32, (CPP, 8, E8), 0) * N_PAD
    sv = s_ref[...][None] + lN
    dv = d_ref[...][None] + lN
    pk_ref[...] = jnp.bitwise_or(sv, dv << 16)


def _tc_prep_pk(src_c, dst_c):
    return pl.pallas_call(
        _pk_body,
        grid=(1,),
        in_specs=[
            pl.BlockSpec((8, E8), lambda i: (0, 0)),
            pl.BlockSpec((8, E8), lambda i: (0, 0)),
        ],
        out_specs=pl.BlockSpec((CPP, 8, E8), lambda i: (0, 0, 0)),
        out_shape=jax.ShapeDtypeStruct((CPP, 8, E8), jnp.int32),
    )(src_c, dst_c)


def _tp_body(part_ref, hs_ref, hd_ref):
    y = part_ref[...]
    hs_ref[...] = y[:, 0, :].T
    hd_ref[...] = y[:, 1, :].T


def _tc_hist_transpose(part):
    return pl.pallas_call(
        _tp_body,
        grid=(1,),
        in_specs=[pl.BlockSpec((NW, 2, N_NODES), lambda i: (0, 0, 0))],
        out_specs=[
            pl.BlockSpec((N_NODES, NW), lambda i: (0, 0)),
            pl.BlockSpec((N_NODES, NW), lambda i: (0, 0)),
        ],
        out_shape=[
            jax.ShapeDtypeStruct((N_NODES, NW), jnp.float32),
            jax.ShapeDtypeStruct((N_NODES, NW), jnp.float32),
        ],
    )(part)


def kernel(x, edge_index, W, b):
    src = edge_index[0]
    dst = edge_index[1]

    e2 = _tc_prep_e2(src.reshape(NW, EPT), dst.reshape(NW, EPT)).reshape(-1)
    zeros2n = jnp.zeros((2 * N_NODES,), jnp.float32)
    part = _sc_degrees(e2, zeros2n).reshape(NW, 2, N_NODES)
    hist_src, hist_dst = _tc_hist_transpose(part)

    featq = _tc_matmul(x, hist_src, W).reshape(-1)
    pk4 = _tc_prep_pk(src.reshape(8, E8), dst.reshape(8, E8)).reshape(-1)
    zeros4 = jnp.zeros((SLAB_P,), jnp.float32)

    aggq = _sc_aggregate(featq, pk4, zeros4)
    agg = aggq.reshape(NQ, CPP, N_PAD)
    return _tc_epilogue(agg, hist_dst, b.reshape(1, D))
